# baseline (speedup 1.0000x reference)
import jax
import jax.numpy as jnp
from jax import lax
from jax.experimental import pallas as pl
from jax.experimental.pallas import tpu as pltpu
from jax.experimental.pallas import tpu_sc as plsc

_E = 1600000
_NC = 2
_NS = 16
_NW = _NC * _NS

_IDX_W = (_E * 2) // _NW
_ICH = 10000
_INCH = _IDX_W // _ICH
_VAL_W = _E // _NW
_VCH = 10000
_VNCH = _VAL_W // _VCH

_INT_MIN = jnp.iinfo(jnp.int32).min


def _stream(src, dst, span0, nch, ch, bufs, in_sems, out_sems, consume=None):
    def sl(c):
        return pl.ds(span0 + c * ch, ch)

    pltpu.async_copy(src.at[sl(0)], bufs[0], in_sems[0])
    for c in range(nch):
        slot = c % 2
        nxt = 1 - slot
        if c + 1 < nch:
            if c >= 1:
                pltpu.make_async_copy(bufs[nxt], dst.at[sl(c - 1)], out_sems[nxt]).wait()
            pltpu.async_copy(src.at[sl(c + 1)], bufs[nxt], in_sems[nxt])
        pltpu.make_async_copy(src.at[sl(c)], bufs[slot], in_sems[slot]).wait()
        if consume is not None:
            consume(bufs[slot])
        pltpu.async_copy(bufs[slot], dst.at[sl(c)], out_sems[slot])
    for c in (nch - 2, nch - 1):
        pltpu.make_async_copy(bufs[c % 2], dst.at[sl(c)], out_sems[c % 2]).wait()


def _sc_body(idx_in, val_in, idx_out, val_out, part_out,
             ibuf0, ibuf1, vbuf0, vbuf1, accbuf, allbuf, partbuf, shared,
             isem0, isem1, osem0, osem1, vsem0, vsem1, wsem0, wsem1):
    cid = lax.axis_index("c")
    sid = lax.axis_index("s")
    wid = cid * _NS + sid

    accbuf[...] = jnp.full((16,), _INT_MIN, jnp.int32)

    def consume(buf):
        def step(t, a):
            return jnp.maximum(a, buf[pl.ds(t * 16, 16)])
        accbuf[...] = lax.fori_loop(0, _ICH // 16, step, accbuf[...])

    _stream(idx_in, idx_out, wid * _IDX_W, _INCH, _ICH,
            (ibuf0, ibuf1), (isem0, isem1), (osem0, osem1), consume)
    _stream(val_in, val_out, wid * _VAL_W, _VNCH, _VCH,
            (vbuf0, vbuf1), (vsem0, vsem1), (wsem0, wsem1))

    pltpu.sync_copy(accbuf, shared.at[sid])
    plsc.subcore_barrier()

    @pl.when(sid == 0)
    def _combine():
        pltpu.sync_copy(shared, allbuf)
        m = allbuf[0]
        for i in range(1, _NS):
            m = jnp.maximum(m, allbuf[i])
        partbuf[...] = m
        pltpu.sync_copy(partbuf, part_out.at[cid])


_sc_kernel = pl.kernel(
    _sc_body,
    out_type=[
        jax.ShapeDtypeStruct((_E * 2,), jnp.int32),
        jax.ShapeDtypeStruct((_E,), jnp.float32),
        jax.ShapeDtypeStruct((_NC, 16), jnp.int32),
    ],
    mesh=plsc.VectorSubcoreMesh(core_axis_name="c", subcore_axis_name="s"),
    compiler_params=pltpu.CompilerParams(
        use_tc_tiling_on_sc=False, needs_layout_passes=False),
    scratch_types=[
        pltpu.VMEM((_ICH,), jnp.int32),
        pltpu.VMEM((_ICH,), jnp.int32),
        pltpu.VMEM((_VCH,), jnp.float32),
        pltpu.VMEM((_VCH,), jnp.float32),
        pltpu.VMEM((16,), jnp.int32),
        pltpu.VMEM((_NS, 16), jnp.int32),
        pltpu.VMEM((16,), jnp.int32),
        pltpu.VMEM_SHARED((_NS, 16), jnp.int32),
        pltpu.SemaphoreType.DMA,
        pltpu.SemaphoreType.DMA,
        pltpu.SemaphoreType.DMA,
        pltpu.SemaphoreType.DMA,
        pltpu.SemaphoreType.DMA,
        pltpu.SemaphoreType.DMA,
        pltpu.SemaphoreType.DMA,
        pltpu.SemaphoreType.DMA,
    ],
)


def _fin_body(part_ref, shape_ref):
    n = jnp.max(part_ref[...]) + 1
    shape_ref[0] = n
    shape_ref[1] = n


def kernel(indices, values):
    idx_flat = (indices.reshape(_E // 128, 128, 2)
                .transpose(0, 2, 1).reshape(_E * 2))
    idx_flat_out, val_out, part = _sc_kernel(idx_flat, values)
    idx_out = (idx_flat_out.reshape(_E // 128, 2, 128)
               .transpose(0, 2, 1).reshape(_E, 2))
    dense_shape = pl.pallas_call(
        _fin_body,
        in_specs=[pl.BlockSpec(memory_space=pltpu.MemorySpace.VMEM)],
        out_specs=pl.BlockSpec(memory_space=pltpu.MemorySpace.SMEM),
        out_shape=jax.ShapeDtypeStruct((2,), jnp.int32),
    )(part)
    return (idx_out, val_out, dense_shape)


# speedup vs baseline: 1.2496x; 1.2496x over previous
import jax
import jax.numpy as jnp
from jax import lax
from jax.experimental import pallas as pl
from jax.experimental.pallas import tpu as pltpu
from jax.experimental.pallas import tpu_sc as plsc

_E = 1600000
_NC = 2
_NS = 16
_NW = _NC * _NS

_IDX_W = (_E * 2) // _NW
_ICH = 20000
_INCH = _IDX_W // _ICH
_VAL_W = _E // _NW
_VCH = 10000
_VNCH = _VAL_W // _VCH

_INT_MIN = jnp.iinfo(jnp.int32).min


def _sc_body(idx_in, val_in, idx_out, val_out, part_out,
             ibuf0, ibuf1, vbuf0, vbuf1, accbuf, allbuf, partbuf, shared,
             isem0, isem1, osem0, osem1, vsem0, vsem1, wsem0, wsem1):
    cid = lax.axis_index("c")
    sid = lax.axis_index("s")
    wid = cid * _NS + sid

    accbuf[...] = jnp.full((16,), _INT_MIN, jnp.int32)

    ibufs, isems, osems = (ibuf0, ibuf1), (isem0, isem1), (osem0, osem1)
    vbufs, vsems, wsems = (vbuf0, vbuf1), (vsem0, vsem1), (wsem0, wsem1)

    def isl(c):
        return pl.ds(wid * _IDX_W + c * _ICH, _ICH)

    def vsl(c):
        return pl.ds(wid * _VAL_W + c * _VCH, _VCH)

    def v_in(c, slot):
        return pltpu.make_async_copy(val_in.at[vsl(c)], vbufs[slot], vsems[slot])

    def v_out(c, slot):
        return pltpu.make_async_copy(vbufs[slot], val_out.at[vsl(c)], wsems[slot])

    def i_in(c, slot):
        return pltpu.make_async_copy(idx_in.at[isl(c)], ibufs[slot], isems[slot])

    def i_out(c, slot):
        return pltpu.make_async_copy(ibufs[slot], idx_out.at[isl(c)], osems[slot])

    i_in(0, 0).start()
    v_in(0, 0).start()
    for c in range(_INCH):
        slot = c % 2
        nxt = 1 - slot
        if c + 1 < _INCH:
            if c >= 1:
                i_out(c - 1, nxt).wait()
                v_out(c - 1, nxt).wait()
            i_in(c + 1, nxt).start()
            v_in(c + 1, nxt).start()
        v_in(c, slot).wait()
        v_out(c, slot).start()
        i_in(c, slot).wait()

        def step(t, a):
            return jnp.maximum(a, ibufs[slot][pl.ds(t * 16, 16)])
        accbuf[...] = lax.fori_loop(0, _ICH // 16, step, accbuf[...],
                                    unroll=8)
        i_out(c, slot).start()
    for c in (_INCH - 2, _INCH - 1):
        i_out(c, c % 2).wait()
        v_out(c, c % 2).wait()

    pltpu.sync_copy(accbuf, shared.at[sid])
    plsc.subcore_barrier()

    @pl.when(sid == 0)
    def _combine():
        pltpu.sync_copy(shared, allbuf)
        m = allbuf[0]
        for i in range(1, _NS):
            m = jnp.maximum(m, allbuf[i])
        partbuf[...] = m
        pltpu.sync_copy(partbuf, part_out.at[cid])


_sc_kernel = pl.kernel(
    _sc_body,
    out_type=[
        jax.ShapeDtypeStruct((_E * 2,), jnp.int32),
        jax.ShapeDtypeStruct((_E,), jnp.float32),
        jax.ShapeDtypeStruct((_NC, 16), jnp.int32),
    ],
    mesh=plsc.VectorSubcoreMesh(core_axis_name="c", subcore_axis_name="s"),
    compiler_params=pltpu.CompilerParams(
        use_tc_tiling_on_sc=False, needs_layout_passes=False),
    scratch_types=[
        pltpu.VMEM((_ICH,), jnp.int32),
        pltpu.VMEM((_ICH,), jnp.int32),
        pltpu.VMEM((_VCH,), jnp.float32),
        pltpu.VMEM((_VCH,), jnp.float32),
        pltpu.VMEM((16,), jnp.int32),
        pltpu.VMEM((_NS, 16), jnp.int32),
        pltpu.VMEM((16,), jnp.int32),
        pltpu.VMEM_SHARED((_NS, 16), jnp.int32),
        pltpu.SemaphoreType.DMA,
        pltpu.SemaphoreType.DMA,
        pltpu.SemaphoreType.DMA,
        pltpu.SemaphoreType.DMA,
        pltpu.SemaphoreType.DMA,
        pltpu.SemaphoreType.DMA,
        pltpu.SemaphoreType.DMA,
        pltpu.SemaphoreType.DMA,
    ],
)


def _fin_body(part_ref, shape_ref):
    n = jnp.max(part_ref[...]) + 1
    shape_ref[0] = n
    shape_ref[1] = n


def kernel(indices, values):
    idx_flat = (indices.reshape(_E // 128, 128, 2)
                .transpose(0, 2, 1).reshape(_E * 2))
    idx_flat_out, val_out, part = _sc_kernel(idx_flat, values)
    idx_out = (idx_flat_out.reshape(_E // 128, 2, 128)
               .transpose(0, 2, 1).reshape(_E, 2))
    dense_shape = pl.pallas_call(
        _fin_body,
        in_specs=[pl.BlockSpec(memory_space=pltpu.MemorySpace.VMEM)],
        out_specs=pl.BlockSpec(memory_space=pltpu.MemorySpace.SMEM),
        out_shape=jax.ShapeDtypeStruct((2,), jnp.int32),
    )(part)
    return (idx_out, val_out, dense_shape)
